# Initial kernel scaffold; baseline (speedup 1.0000x reference)
import jax
import jax.numpy as jnp
from jax.experimental import pallas as pl


def kernel(stu_emb, exer_emb, kn_emb, W_sfe, a_sfe, W_efs, a_efs, W_efk, a_efk, W_kfe, a_kfe, Wa_s0, ba_s0, Wa_e0, ba_e0, Wa_e1, ba_e1, Wa_k0, ba_k0, s_from_e_edge_index, e_from_s_edge_index, e_from_k_edge_index, k_from_e_edge_index):
    raise NotImplementedError("write your pallas kernel here")


# speedup vs baseline: 5.8256x; 5.8256x over previous
import functools

import jax
import jax.numpy as jnp
from jax import lax
from jax.experimental import pallas as pl
from jax.experimental.pallas import tpu as pltpu
from jax.experimental.pallas import tpu_sc as plsc

S_N, X_N, KD = 40000, 10000, 128

SE_N, SE_NP, SE_E, SE_EP = 50000, 50176, 500000, 512000
EK_N, EK_NP, EK_E, EK_EP = 10128, 10240, 160000, 163840

N_TILES = 16
N_CORES = 2
EB = 128


def _matmul_pair_body(h_ref, w1t_ref, a1_ref, w2t_ref, a2_ref,
                      v1_ref, u1_ref, v2_ref, u2_ref):
    h = h_ref[...]
    z1 = jnp.dot(h, w1t_ref[...], preferred_element_type=jnp.float32)
    u1 = jnp.exp(jnp.dot(z1, a1_ref[...], preferred_element_type=jnp.float32))
    v1_ref[...] = z1 * u1
    u1_ref[...] = u1
    z2 = jnp.dot(h, w2t_ref[...], preferred_element_type=jnp.float32)
    u2 = jnp.exp(jnp.dot(z2, a2_ref[...], preferred_element_type=jnp.float32))
    v2_ref[...] = z2 * u2
    u2_ref[...] = u2


def _matmul_pair(h, w1t, a1, w2t, a2, bn=512):
    np_ = h.shape[0]
    grid = (np_ // bn,)
    full = pl.BlockSpec((KD, KD), lambda i: (0, 0))
    fulla = pl.BlockSpec((KD, 1), lambda i: (0, 0))
    rows = pl.BlockSpec((bn, KD), lambda i: (i, 0))
    us = pl.BlockSpec((bn, 1), lambda i: (i, 0))
    return pl.pallas_call(
        _matmul_pair_body,
        grid=grid,
        in_specs=[rows, full, fulla, full, fulla],
        out_specs=[rows, us, rows, us],
        out_shape=[
            jax.ShapeDtypeStruct((np_, KD), jnp.float32),
            jax.ShapeDtypeStruct((np_, 1), jnp.float32),
            jax.ShapeDtypeStruct((np_, KD), jnp.float32),
            jax.ShapeDtypeStruct((np_, 1), jnp.float32),
        ],
    )(h, w1t, a1, w2t, a2)


def _make_sc_edge_kernel(np_, ep, nch, cw):
    et = ep // (N_CORES * N_TILES)
    nb = et // EB
    rpt = np_ // N_TILES
    zr = 98 if cw == 32 else 40
    reps = rpt // zr
    assert et % EB == 0 and np_ % N_TILES == 0 and rpt % zr == 0

    mesh = plsc.VectorSubcoreMesh(core_axis_name="c", subcore_axis_name="s")

    @functools.partial(
        pl.kernel,
        out_type=[
            jax.ShapeDtypeStruct((N_CORES, nch, np_, cw), jnp.float32),
            jax.ShapeDtypeStruct((N_CORES, N_TILES, 1, rpt), jnp.float32),
        ],
        mesh=mesh,
        compiler_params=pltpu.CompilerParams(
            needs_layout_passes=False, use_tc_tiling_on_sc=False),
        scratch_types=[
            pltpu.VMEM((EB, cw), jnp.float32),
            pltpu.VMEM((zr, cw), jnp.float32),
            pltpu.VMEM((rpt,), jnp.float32),
            pltpu.VMEM((EB,), jnp.int32),
            pltpu.VMEM((1, EB), jnp.int32),
            pltpu.VMEM((EB,), jnp.float32),
            pltpu.VMEM_SHARED((np_, cw), jnp.float32),
            pltpu.VMEM_SHARED((np_,), jnp.float32),
            pltpu.SemaphoreType.DMA,
        ],
    )
    def sc_edge(v4, u, src, dst, part, den,
                rows, zacc, zden, gidx, sidx, ubuf, acc_sp, den_sp, sem):
        core = lax.axis_index("c")
        sub = lax.axis_index("s")
        wid = core * N_TILES + sub
        tbase = wid * et
        row0 = sub * rpt
        fz = jnp.zeros((16,), jnp.float32)

        @pl.loop(0, zr)
        def _(r):
            for kk in range(cw // 16):
                zacc[r, pl.ds(16 * kk, 16)] = fz

        @pl.loop(0, rpt // 16)
        def _(i):
            zden[pl.ds(16 * i, 16)] = fz

        pltpu.sync_copy(zden, den_sp.at[pl.ds(row0, rpt)])
        plsc.subcore_barrier()

        @pl.loop(0, nb)
        def _(b):
            off = tbase + b * EB
            pltpu.sync_copy(src.at[pl.ds(off, EB)], gidx)
            pltpu.sync_copy(dst.at[pl.ds(off, EB)], sidx.at[0])
            pltpu.async_copy(u.at[gidx], ubuf, sem).wait()
            pltpu.sync_copy(ubuf, den_sp.at[sidx.at[0]], add=True)

        plsc.subcore_barrier()
        pltpu.sync_copy(den_sp.at[pl.ds(row0, rpt)], den.at[core, sub, 0])

        for c in range(nch):
            @pl.loop(0, reps)
            def _(kk):
                pltpu.sync_copy(zacc, acc_sp.at[pl.ds(row0 + kk * zr, zr)])

            plsc.subcore_barrier()

            @pl.loop(0, nb)
            def _(b):
                off = tbase + b * EB
                pltpu.sync_copy(src.at[pl.ds(off, EB)], gidx)
                pltpu.sync_copy(dst.at[pl.ds(off, EB)], sidx.at[0])
                if nch > 1:
                    for k in range(EB // 16):
                        sl = pl.ds(16 * k, 16)
                        gidx[sl] = gidx[sl] * nch + c
                pltpu.async_copy(v4.at[gidx], rows, sem).wait()
                pltpu.sync_copy(rows, acc_sp.at[sidx.at[0]], add=True)

            plsc.subcore_barrier()
            pltpu.sync_copy(acc_sp.at[pl.ds(row0, rpt)],
                            part.at[core, c, pl.ds(row0, rpt)])
            plsc.subcore_barrier()

    return sc_edge


_sc_edge_se = _make_sc_edge_kernel(SE_NP, SE_EP, 4, 32)
_sc_edge_ek = _make_sc_edge_kernel(EK_NP, EK_EP, 1, KD)


def _conv_from_chunks(part_ref, den_ref):
    pp = part_ref[...]
    nch = pp.shape[1]
    num = pp[0] + pp[1]
    full = jnp.concatenate([num[c] for c in range(nch)], axis=-1) \
        if nch > 1 else num[0]
    dd = den_ref[...]
    dsum = dd[0] + dd[1]
    dsafe = jnp.where(dsum == 0.0, 1.0, dsum)
    return full / dsafe


def _combine_add_body(emb_ref, part_ref, den_ref, out_ref):
    out_ref[...] = emb_ref[...] + _conv_from_chunks(part_ref, den_ref)


def _combine_add(emb, part, den3, row_off, bn):
    n = emb.shape[0]
    nch, cw = part.shape[1], part.shape[3]
    grid = (n // bn,)
    ob = row_off // bn
    return pl.pallas_call(
        _combine_add_body,
        grid=grid,
        in_specs=[
            pl.BlockSpec((bn, KD), lambda i: (i, 0)),
            pl.BlockSpec((2, nch, bn, cw), lambda i, ob=ob: (0, 0, ob + i, 0)),
            pl.BlockSpec((2, bn, 1), lambda i, ob=ob: (0, ob + i, 0)),
        ],
        out_specs=pl.BlockSpec((bn, KD), lambda i: (i, 0)),
        out_shape=jax.ShapeDtypeStruct((n, KD), jnp.float32),
    )(emb, part, den3)


def _combine_exer_body(emb_ref, p0_ref, d0_ref, p1_ref, d1_ref,
                       wa0_ref, wa1_ref, ba0_ref, ba1_ref, out_ref):
    emb = emb_ref[...]
    c0 = _conv_from_chunks(p0_ref, d0_ref)
    c1 = _conv_from_chunks(p1_ref, d1_ref)
    wa0 = wa0_ref[...]
    wa1 = wa1_ref[...]
    s0 = (jnp.sum(emb * wa0[0:1, :], axis=1, keepdims=True)
          + jnp.sum(c0 * wa0[1:2, :], axis=1, keepdims=True) + ba0_ref[0])
    s1 = (jnp.sum(emb * wa1[0:1, :], axis=1, keepdims=True)
          + jnp.sum(c1 * wa1[1:2, :], axis=1, keepdims=True) + ba1_ref[0])
    m = jnp.maximum(s0, s1)
    e0 = jnp.exp(s0 - m)
    e1 = jnp.exp(s1 - m)
    out_ref[...] = emb + (e0 * c0 + e1 * c1) / (e0 + e1)


def _combine_exer(emb, p0, d0, off0, p1, d1, off1, wa0, wa1, ba0, ba1, bn=400):
    n = emb.shape[0]
    nch0, cw0 = p0.shape[1], p0.shape[3]
    nch1, cw1 = p1.shape[1], p1.shape[3]
    grid = (n // bn,)
    ob0 = off0 // bn
    ob1 = off1 // bn
    return pl.pallas_call(
        _combine_exer_body,
        grid=grid,
        in_specs=[
            pl.BlockSpec((bn, KD), lambda i: (i, 0)),
            pl.BlockSpec((2, nch0, bn, cw0), lambda i: (0, 0, ob0 + i, 0)),
            pl.BlockSpec((2, bn, 1), lambda i: (0, ob0 + i, 0)),
            pl.BlockSpec((2, nch1, bn, cw1), lambda i: (0, 0, ob1 + i, 0)),
            pl.BlockSpec((2, bn, 1), lambda i: (0, ob1 + i, 0)),
            pl.BlockSpec((2, KD), lambda i: (0, 0)),
            pl.BlockSpec((2, KD), lambda i: (0, 0)),
            pl.BlockSpec(memory_space=pltpu.SMEM),
            pl.BlockSpec(memory_space=pltpu.SMEM),
        ],
        out_specs=pl.BlockSpec((bn, KD), lambda i: (i, 0)),
        out_shape=jax.ShapeDtypeStruct((n, KD), jnp.float32),
    )(emb, p0, d0, p1, d1, wa0, wa1, ba0, ba1)


def _pad_rows(x, np_):
    return jnp.pad(x, ((0, np_ - x.shape[0]), (0, 0)))


def _pad_edges(edge_index, ep, n):
    e = edge_index.shape[1]
    src = jnp.pad(edge_index[0], (0, ep - e))
    dst = jnp.pad(edge_index[1], (0, ep - e), constant_values=n)
    return src, dst


def _acol(a):
    return a[:KD].reshape(KD, 1)


def kernel(stu_emb, exer_emb, kn_emb, W_sfe, a_sfe, W_efs, a_efs, W_efk,
           a_efk, W_kfe, a_kfe, Wa_s0, ba_s0, Wa_e0, ba_e0, Wa_e1, ba_e1,
           Wa_k0, ba_k0, s_from_e_edge_index, e_from_s_edge_index,
           e_from_k_edge_index, k_from_e_edge_index):
    h_se = _pad_rows(jnp.concatenate([exer_emb, stu_emb], axis=0), SE_NP)
    h_ek = _pad_rows(jnp.concatenate([exer_emb, kn_emb], axis=0), EK_NP)

    v_sfe, u_sfe, v_efs, u_efs = _matmul_pair(
        h_se, W_sfe.T, _acol(a_sfe), W_efs.T, _acol(a_efs))
    v_efk, u_efk, v_kfe, u_kfe = _matmul_pair(
        h_ek, W_efk.T, _acol(a_efk), W_kfe.T, _acol(a_kfe))

    src_sfe, dst_sfe = _pad_edges(s_from_e_edge_index, SE_EP, SE_N)
    src_efs, dst_efs = _pad_edges(e_from_s_edge_index, SE_EP, SE_N)
    src_efk, dst_efk = _pad_edges(e_from_k_edge_index, EK_EP, EK_N)
    src_kfe, dst_kfe = _pad_edges(k_from_e_edge_index, EK_EP, EK_N)

    part_sfe, den_sfe = _sc_edge_se(
        v_sfe.reshape(SE_NP * 4, 32), u_sfe.reshape(SE_NP),
        src_sfe, dst_sfe)
    part_efs, den_efs = _sc_edge_se(
        v_efs.reshape(SE_NP * 4, 32), u_efs.reshape(SE_NP),
        src_efs, dst_efs)
    part_efk, den_efk = _sc_edge_ek(
        v_efk, u_efk.reshape(EK_NP), src_efk, dst_efk)
    part_kfe, den_kfe = _sc_edge_ek(
        v_kfe, u_kfe.reshape(EK_NP), src_kfe, dst_kfe)

    den_sfe3 = den_sfe.reshape(2, SE_NP, 1)
    den_efs3 = den_efs.reshape(2, SE_NP, 1)
    den_efk3 = den_efk.reshape(2, EK_NP, 1)
    den_kfe3 = den_kfe.reshape(2, EK_NP, 1)

    ult_stu = _combine_add(stu_emb, part_sfe, den_sfe3, 0, 400)
    ult_kn = _combine_add(kn_emb, part_kfe, den_kfe3, X_N, 16)
    ult_exer = _combine_exer(
        exer_emb, part_efs, den_efs3, S_N, part_efk, den_efk3, 0,
        Wa_e0.reshape(2, KD), Wa_e1.reshape(2, KD), ba_e0, ba_e1)

    return (ult_stu, ult_exer, ult_kn)


# speedup vs baseline: 7.1424x; 1.2260x over previous
import functools

import jax
import jax.numpy as jnp
from jax import lax
from jax.experimental import pallas as pl
from jax.experimental.pallas import tpu as pltpu
from jax.experimental.pallas import tpu_sc as plsc

S_N, X_N, KD = 40000, 10000, 128
VW = 144

SE_N, SE_NP, SE_E, SE_EP = 50000, 50176, 500000, 512000
EK_N, EK_NP, EK_E, EK_EP = 10128, 10240, 160000, 163840
SE_AN = 40960
FU_AN = 10240

N_TILES = 16
N_CORES = 2
EB = 128


def _mm_se_body(h_ref, w1t_ref, a1_ref, w2t_ref, a2_ref,
                v1_ref, u1_ref, vu2_ref):
    h = h_ref[...]
    z1 = jnp.dot(h, w1t_ref[...], preferred_element_type=jnp.float32)
    u1 = jnp.exp(jnp.dot(z1, a1_ref[...], preferred_element_type=jnp.float32))
    v1_ref[...] = z1 * u1
    u1_ref[...] = u1
    z2 = jnp.dot(h, w2t_ref[...], preferred_element_type=jnp.float32)
    u2 = jnp.exp(jnp.dot(z2, a2_ref[...], preferred_element_type=jnp.float32))
    vu2_ref[...] = jnp.concatenate(
        [z2 * u2, u2, jnp.zeros((h.shape[0], VW - KD - 1), jnp.float32)],
        axis=1)


def _mm_se(h, w1t, a1, w2t, a2, bn=512):
    np_ = h.shape[0]
    grid = (np_ // bn,)
    full = pl.BlockSpec((KD, KD), lambda i: (0, 0))
    fulla = pl.BlockSpec((KD, 1), lambda i: (0, 0))
    rows = pl.BlockSpec((bn, KD), lambda i: (i, 0))
    us = pl.BlockSpec((bn, 1), lambda i: (i, 0))
    vrows = pl.BlockSpec((bn, VW), lambda i: (i, 0))
    return pl.pallas_call(
        _mm_se_body,
        grid=grid,
        in_specs=[rows, full, fulla, full, fulla],
        out_specs=[rows, us, vrows],
        out_shape=[
            jax.ShapeDtypeStruct((np_, KD), jnp.float32),
            jax.ShapeDtypeStruct((np_, 1), jnp.float32),
            jax.ShapeDtypeStruct((np_, VW), jnp.float32),
        ],
    )(h, w1t, a1, w2t, a2)


def _mm_ek_body(h_ref, w1t_ref, a1_ref, w2t_ref, a2_ref, vu1_ref, vu2_ref):
    h = h_ref[...]
    zp = jnp.zeros((h.shape[0], VW - KD - 1), jnp.float32)
    z1 = jnp.dot(h, w1t_ref[...], preferred_element_type=jnp.float32)
    u1 = jnp.exp(jnp.dot(z1, a1_ref[...], preferred_element_type=jnp.float32))
    vu1_ref[...] = jnp.concatenate([z1 * u1, u1, zp], axis=1)
    z2 = jnp.dot(h, w2t_ref[...], preferred_element_type=jnp.float32)
    u2 = jnp.exp(jnp.dot(z2, a2_ref[...], preferred_element_type=jnp.float32))
    vu2_ref[...] = jnp.concatenate([z2 * u2, u2, zp], axis=1)


def _mm_ek(h, w1t, a1, w2t, a2, bn=512):
    np_ = h.shape[0]
    grid = (np_ // bn,)
    full = pl.BlockSpec((KD, KD), lambda i: (0, 0))
    fulla = pl.BlockSpec((KD, 1), lambda i: (0, 0))
    rows = pl.BlockSpec((bn, KD), lambda i: (i, 0))
    vrows = pl.BlockSpec((bn, VW), lambda i: (i, 0))
    return pl.pallas_call(
        _mm_ek_body,
        grid=grid,
        in_specs=[rows, full, fulla, full, fulla],
        out_specs=[vrows, vrows],
        out_shape=[
            jax.ShapeDtypeStruct((np_, VW), jnp.float32),
            jax.ShapeDtypeStruct((np_, VW), jnp.float32),
        ],
    )(h, w1t, a1, w2t, a2)


def _make_sc_edge_kernel(an_, ep, nch, cw):
    et = ep // (N_CORES * N_TILES)
    nb = et // EB
    rpt = an_ // N_TILES
    zr = 80
    reps = rpt // zr
    assert et % EB == 0 and an_ % N_TILES == 0 and rpt % zr == 0

    mesh = plsc.VectorSubcoreMesh(core_axis_name="c", subcore_axis_name="s")

    @functools.partial(
        pl.kernel,
        out_type=[
            jax.ShapeDtypeStruct((N_CORES, nch, an_, cw), jnp.float32),
            jax.ShapeDtypeStruct((N_CORES, N_TILES, 1, rpt), jnp.float32),
        ],
        mesh=mesh,
        compiler_params=pltpu.CompilerParams(
            needs_layout_passes=False, use_tc_tiling_on_sc=False),
        scratch_types=[
            pltpu.VMEM((EB, cw), jnp.float32),
            pltpu.VMEM((zr, cw), jnp.float32),
            pltpu.VMEM((rpt,), jnp.float32),
            pltpu.VMEM((EB,), jnp.int32),
            pltpu.VMEM((1, EB), jnp.int32),
            pltpu.VMEM((EB,), jnp.float32),
            pltpu.VMEM_SHARED((an_, cw), jnp.float32),
            pltpu.VMEM_SHARED((an_,), jnp.float32),
            pltpu.SemaphoreType.DMA,
        ],
    )
    def sc_edge(v4, u, src, dst, part, den,
                rows, zacc, zden, gidx, sidx, ubuf, acc_sp, den_sp, sem):
        core = lax.axis_index("c")
        sub = lax.axis_index("s")
        wid = core * N_TILES + sub
        tbase = wid * et
        row0 = sub * rpt
        fz = jnp.zeros((16,), jnp.float32)

        @pl.loop(0, zr)
        def _(r):
            for kk in range(cw // 16):
                zacc[r, pl.ds(16 * kk, 16)] = fz

        @pl.loop(0, rpt // 16)
        def _(i):
            zden[pl.ds(16 * i, 16)] = fz

        pltpu.sync_copy(zden, den_sp.at[pl.ds(row0, rpt)])
        plsc.subcore_barrier()

        @pl.loop(0, nb)
        def _(b):
            off = tbase + b * EB
            pltpu.sync_copy(src.at[pl.ds(off, EB)], gidx)
            pltpu.sync_copy(dst.at[pl.ds(off, EB)], sidx.at[0])
            pltpu.async_copy(u.at[gidx], ubuf, sem).wait()
            pltpu.sync_copy(ubuf, den_sp.at[sidx.at[0]], add=True)

        plsc.subcore_barrier()
        pltpu.sync_copy(den_sp.at[pl.ds(row0, rpt)], den.at[core, sub, 0])

        for c in range(nch):
            @pl.loop(0, reps)
            def _(kk):
                pltpu.sync_copy(zacc, acc_sp.at[pl.ds(row0 + kk * zr, zr)])

            plsc.subcore_barrier()

            @pl.loop(0, nb)
            def _(b):
                off = tbase + b * EB
                pltpu.sync_copy(src.at[pl.ds(off, EB)], gidx)
                pltpu.sync_copy(dst.at[pl.ds(off, EB)], sidx.at[0])
                if nch > 1:
                    for k in range(EB // 16):
                        sl = pl.ds(16 * k, 16)
                        gidx[sl] = gidx[sl] * nch + c
                pltpu.async_copy(v4.at[gidx], rows, sem).wait()
                pltpu.sync_copy(rows, acc_sp.at[sidx.at[0]], add=True)

            plsc.subcore_barrier()
            pltpu.sync_copy(acc_sp.at[pl.ds(row0, rpt)],
                            part.at[core, c, pl.ds(row0, rpt)])
            plsc.subcore_barrier()

    return sc_edge


def _make_sc_fused_kernel(an_, ep):
    et = ep // (N_CORES * N_TILES)
    nb = et // EB
    rpt = an_ // N_TILES
    zr = 40
    reps = rpt // zr
    assert et % EB == 0 and an_ % N_TILES == 0 and rpt % zr == 0

    mesh = plsc.VectorSubcoreMesh(core_axis_name="c", subcore_axis_name="s")

    @functools.partial(
        pl.kernel,
        out_type=jax.ShapeDtypeStruct((N_CORES, 1, an_, VW), jnp.float32),
        mesh=mesh,
        compiler_params=pltpu.CompilerParams(
            needs_layout_passes=False, use_tc_tiling_on_sc=False),
        scratch_types=[
            pltpu.VMEM((EB, VW), jnp.float32),
            pltpu.VMEM((zr, VW), jnp.float32),
            pltpu.VMEM((EB,), jnp.int32),
            pltpu.VMEM((1, EB), jnp.int32),
            pltpu.VMEM_SHARED((an_, VW), jnp.float32),
            pltpu.SemaphoreType.DMA,
        ],
    )
    def sc_fused(vu, src, dst, part, rows, zacc, gidx, sidx, acc_sp, sem):
        core = lax.axis_index("c")
        sub = lax.axis_index("s")
        tbase = (core * N_TILES + sub) * et
        row0 = sub * rpt
        fz = jnp.zeros((16,), jnp.float32)

        @pl.loop(0, zr)
        def _(r):
            for kk in range(VW // 16):
                zacc[r, pl.ds(16 * kk, 16)] = fz

        @pl.loop(0, reps)
        def _(kk):
            pltpu.sync_copy(zacc, acc_sp.at[pl.ds(row0 + kk * zr, zr)])

        plsc.subcore_barrier()

        @pl.loop(0, nb)
        def _(b):
            off = tbase + b * EB
            pltpu.sync_copy(src.at[pl.ds(off, EB)], gidx)
            pltpu.sync_copy(dst.at[pl.ds(off, EB)], sidx.at[0])
            pltpu.async_copy(vu.at[gidx], rows, sem).wait()
            pltpu.sync_copy(rows, acc_sp.at[sidx.at[0]], add=True)

        plsc.subcore_barrier()
        pltpu.sync_copy(acc_sp.at[pl.ds(row0, rpt)],
                        part.at[core, 0, pl.ds(row0, rpt)])

    return sc_fused


_sc_edge_se = _make_sc_edge_kernel(SE_AN, SE_EP, 4, 32)
_sc_fused_se = _make_sc_fused_kernel(FU_AN, SE_EP)
_sc_fused_ek = _make_sc_fused_kernel(FU_AN, EK_EP)


def _conv_from_chunks(part_ref, den_ref):
    pp = part_ref[...]
    nch = pp.shape[1]
    num = pp[0] + pp[1]
    full = jnp.concatenate([num[c] for c in range(nch)], axis=-1) \
        if nch > 1 else num[0]
    dd = den_ref[...]
    dsum = dd[0] + dd[1]
    dsafe = jnp.where(dsum == 0.0, 1.0, dsum)
    return full / dsafe


def _conv_fused(part_ref):
    pp = part_ref[...]
    s = pp[0, 0] + pp[1, 0]
    num = s[:, :KD]
    den = s[:, KD:KD + 1]
    return num / jnp.where(den == 0.0, 1.0, den)


def _combine_add_body(emb_ref, part_ref, den_ref, out_ref):
    out_ref[...] = emb_ref[...] + _conv_from_chunks(part_ref, den_ref)


def _combine_add(emb, part, den3, row_off, bn):
    n = emb.shape[0]
    nch, cw = part.shape[1], part.shape[3]
    grid = (n // bn,)
    ob = row_off // bn
    return pl.pallas_call(
        _combine_add_body,
        grid=grid,
        in_specs=[
            pl.BlockSpec((bn, KD), lambda i: (i, 0)),
            pl.BlockSpec((2, nch, bn, cw), lambda i, ob=ob: (0, 0, ob + i, 0)),
            pl.BlockSpec((2, bn, 1), lambda i, ob=ob: (0, ob + i, 0)),
        ],
        out_specs=pl.BlockSpec((bn, KD), lambda i: (i, 0)),
        out_shape=jax.ShapeDtypeStruct((n, KD), jnp.float32),
    )(emb, part, den3)


def _combine_add_fused_body(emb_ref, part_ref, out_ref):
    out_ref[...] = emb_ref[...] + _conv_fused(part_ref)


def _combine_add_fused(emb, part, row_off, bn):
    n = emb.shape[0]
    grid = (n // bn,)
    ob = row_off // bn
    return pl.pallas_call(
        _combine_add_fused_body,
        grid=grid,
        in_specs=[
            pl.BlockSpec((bn, KD), lambda i: (i, 0)),
            pl.BlockSpec((2, 1, bn, VW), lambda i, ob=ob: (0, 0, ob + i, 0)),
        ],
        out_specs=pl.BlockSpec((bn, KD), lambda i: (i, 0)),
        out_shape=jax.ShapeDtypeStruct((n, KD), jnp.float32),
    )(emb, part)


def _combine_exer_body(emb_ref, p0_ref, p1_ref,
                       wa0_ref, wa1_ref, ba0_ref, ba1_ref, out_ref):
    emb = emb_ref[...]
    c0 = _conv_fused(p0_ref)
    c1 = _conv_fused(p1_ref)
    wa0 = wa0_ref[...]
    wa1 = wa1_ref[...]
    s0 = (jnp.sum(emb * wa0[0:1, :], axis=1, keepdims=True)
          + jnp.sum(c0 * wa0[1:2, :], axis=1, keepdims=True) + ba0_ref[0])
    s1 = (jnp.sum(emb * wa1[0:1, :], axis=1, keepdims=True)
          + jnp.sum(c1 * wa1[1:2, :], axis=1, keepdims=True) + ba1_ref[0])
    m = jnp.maximum(s0, s1)
    e0 = jnp.exp(s0 - m)
    e1 = jnp.exp(s1 - m)
    out_ref[...] = emb + (e0 * c0 + e1 * c1) / (e0 + e1)


def _combine_exer(emb, p0, p1, wa0, wa1, ba0, ba1, bn=400):
    n = emb.shape[0]
    grid = (n // bn,)
    return pl.pallas_call(
        _combine_exer_body,
        grid=grid,
        in_specs=[
            pl.BlockSpec((bn, KD), lambda i: (i, 0)),
            pl.BlockSpec((2, 1, bn, VW), lambda i: (0, 0, i, 0)),
            pl.BlockSpec((2, 1, bn, VW), lambda i: (0, 0, i, 0)),
            pl.BlockSpec((2, KD), lambda i: (0, 0)),
            pl.BlockSpec((2, KD), lambda i: (0, 0)),
            pl.BlockSpec(memory_space=pltpu.SMEM),
            pl.BlockSpec(memory_space=pltpu.SMEM),
        ],
        out_specs=pl.BlockSpec((bn, KD), lambda i: (i, 0)),
        out_shape=jax.ShapeDtypeStruct((n, KD), jnp.float32),
    )(emb, p0, p1, wa0, wa1, ba0, ba1)


def _pad_rows(x, np_):
    return jnp.pad(x, ((0, np_ - x.shape[0]), (0, 0)))


def _pad1(x, ep, cv=0):
    return jnp.pad(x, (0, ep - x.shape[0]), constant_values=cv)


def _acol(a):
    return a[:KD].reshape(KD, 1)


def kernel(stu_emb, exer_emb, kn_emb, W_sfe, a_sfe, W_efs, a_efs, W_efk,
           a_efk, W_kfe, a_kfe, Wa_s0, ba_s0, Wa_e0, ba_e0, Wa_e1, ba_e1,
           Wa_k0, ba_k0, s_from_e_edge_index, e_from_s_edge_index,
           e_from_k_edge_index, k_from_e_edge_index):
    h_se = _pad_rows(jnp.concatenate([exer_emb, stu_emb], axis=0), SE_NP)
    h_ek = _pad_rows(jnp.concatenate([exer_emb, kn_emb], axis=0), EK_NP)

    v_sfe, u_sfe, vu_efs = _mm_se(
        h_se, W_sfe.T, _acol(a_sfe), W_efs.T, _acol(a_efs))
    vu_efk, vu_kfe = _mm_ek(
        h_ek, W_efk.T, _acol(a_efk), W_kfe.T, _acol(a_kfe))

    src_sfe = _pad1(s_from_e_edge_index[0], SE_EP)
    dst_sfe = _pad1(jnp.minimum(s_from_e_edge_index[1], S_N), SE_EP, S_N)
    src_efs = _pad1(e_from_s_edge_index[0], SE_EP)
    d = e_from_s_edge_index[1]
    dst_efs = _pad1(jnp.where(d >= S_N, d - S_N, X_N), SE_EP, X_N)
    src_efk = _pad1(e_from_k_edge_index[0], EK_EP)
    dst_efk = _pad1(e_from_k_edge_index[1], EK_EP, EK_N)
    src_kfe = _pad1(k_from_e_edge_index[0], EK_EP)
    dst_kfe = _pad1(k_from_e_edge_index[1], EK_EP, EK_N)

    part_sfe, den_sfe = _sc_edge_se(
        v_sfe.reshape(SE_NP * 4, 32), u_sfe.reshape(SE_NP),
        src_sfe, dst_sfe)
    part_efs = _sc_fused_se(vu_efs, src_efs, dst_efs)
    part_efk = _sc_fused_ek(vu_efk, src_efk, dst_efk)
    part_kfe = _sc_fused_ek(vu_kfe, src_kfe, dst_kfe)

    den_sfe3 = den_sfe.reshape(2, SE_AN, 1)

    ult_stu = _combine_add(stu_emb, part_sfe, den_sfe3, 0, 400)
    ult_kn = _combine_add_fused(kn_emb, part_kfe, X_N, 16)
    ult_exer = _combine_exer(
        exer_emb, part_efs, part_efk,
        Wa_e0.reshape(2, KD), Wa_e1.reshape(2, KD), ba_e0, ba_e1)

    return (ult_stu, ult_exer, ult_kn)


# speedup vs baseline: 7.9463x; 1.1126x over previous
import functools

import jax
import jax.numpy as jnp
from jax import lax
from jax.experimental import pallas as pl
from jax.experimental.pallas import tpu as pltpu
from jax.experimental.pallas import tpu_sc as plsc

S_N, X_N, KD = 40000, 10000, 128
VW = 144
CW = 48

SE_N, SE_NP, SE_E, SE_EP = 50000, 50176, 500000, 512000
EK_N, EK_NP, EK_E, EK_EP = 10128, 10240, 160000, 163840
SE_AN = 40960
FU_AN = 10240

N_TILES = 16
N_CORES = 2
EB = 128


def _mm_pair_body(h_ref, w1t_ref, a1_ref, w2t_ref, a2_ref, vu1_ref, vu2_ref):
    h = h_ref[...]
    zp = jnp.zeros((h.shape[0], VW - KD - 1), jnp.float32)
    z1 = jnp.dot(h, w1t_ref[...], preferred_element_type=jnp.float32)
    u1 = jnp.exp(jnp.dot(z1, a1_ref[...], preferred_element_type=jnp.float32))
    vu1_ref[...] = jnp.concatenate([z1 * u1, u1, zp], axis=1)
    z2 = jnp.dot(h, w2t_ref[...], preferred_element_type=jnp.float32)
    u2 = jnp.exp(jnp.dot(z2, a2_ref[...], preferred_element_type=jnp.float32))
    vu2_ref[...] = jnp.concatenate([z2 * u2, u2, zp], axis=1)


def _mm_pair(h, w1t, a1, w2t, a2, bn=512):
    np_ = h.shape[0]
    grid = (np_ // bn,)
    full = pl.BlockSpec((KD, KD), lambda i: (0, 0))
    fulla = pl.BlockSpec((KD, 1), lambda i: (0, 0))
    rows = pl.BlockSpec((bn, KD), lambda i: (i, 0))
    vrows = pl.BlockSpec((bn, VW), lambda i: (i, 0))
    return pl.pallas_call(
        _mm_pair_body,
        grid=grid,
        in_specs=[rows, full, fulla, full, fulla],
        out_specs=[vrows, vrows],
        out_shape=[
            jax.ShapeDtypeStruct((np_, VW), jnp.float32),
            jax.ShapeDtypeStruct((np_, VW), jnp.float32),
        ],
    )(h, w1t, a1, w2t, a2)


def _make_sc_chunk_kernel(an_, ep):
    et = ep // (N_CORES * N_TILES)
    nb = et // EB
    rpt = an_ // N_TILES
    assert et % EB == 0 and an_ % N_TILES == 0 and rpt % EB == 0

    mesh = plsc.VectorSubcoreMesh(core_axis_name="c", subcore_axis_name="s")

    @functools.partial(
        pl.kernel,
        out_type=jax.ShapeDtypeStruct((N_CORES, 3, an_, CW), jnp.float32),
        mesh=mesh,
        compiler_params=pltpu.CompilerParams(
            needs_layout_passes=False, use_tc_tiling_on_sc=False),
        scratch_types=[
            pltpu.VMEM((EB, CW), jnp.float32),
            pltpu.VMEM((EB,), jnp.int32),
            pltpu.VMEM((1, EB), jnp.int32),
            pltpu.VMEM_SHARED((an_, CW), jnp.float32),
            pltpu.SemaphoreType.DMA,
        ],
    )
    def sc_chunk(vu3, src, dst, part, rows, gidx, sidx, acc_sp, sem):
        core = lax.axis_index("c")
        sub = lax.axis_index("s")
        tbase = (core * N_TILES + sub) * et
        row0 = sub * rpt
        fz = jnp.zeros((16,), jnp.float32)

        for c in range(3):
            @pl.loop(0, EB)
            def _(r):
                for kk in range(CW // 16):
                    rows[r, pl.ds(16 * kk, 16)] = fz

            @pl.loop(0, rpt // EB)
            def _(kk):
                pltpu.sync_copy(rows, acc_sp.at[pl.ds(row0 + kk * EB, EB)])

            plsc.subcore_barrier()

            @pl.loop(0, nb)
            def _(b):
                off = tbase + b * EB
                pltpu.sync_copy(src.at[pl.ds(off, EB)], gidx)
                pltpu.sync_copy(dst.at[pl.ds(off, EB)], sidx.at[0])
                for k in range(EB // 16):
                    sl = pl.ds(16 * k, 16)
                    gidx[sl] = gidx[sl] * 3 + c
                pltpu.async_copy(vu3.at[gidx], rows, sem).wait()
                pltpu.sync_copy(rows, acc_sp.at[sidx.at[0]], add=True)

            plsc.subcore_barrier()
            pltpu.sync_copy(acc_sp.at[pl.ds(row0, rpt)],
                            part.at[core, c, pl.ds(row0, rpt)])
            plsc.subcore_barrier()

    return sc_chunk


def _make_sc_fused_kernel(an_, ep):
    et = ep // (N_CORES * N_TILES)
    nb = et // EB
    rpt = an_ // N_TILES
    zr = 40
    reps = rpt // zr
    assert et % EB == 0 and an_ % N_TILES == 0 and rpt % zr == 0

    mesh = plsc.VectorSubcoreMesh(core_axis_name="c", subcore_axis_name="s")

    @functools.partial(
        pl.kernel,
        out_type=jax.ShapeDtypeStruct((N_CORES, 1, an_, VW), jnp.float32),
        mesh=mesh,
        compiler_params=pltpu.CompilerParams(
            needs_layout_passes=False, use_tc_tiling_on_sc=False),
        scratch_types=[
            pltpu.VMEM((EB, VW), jnp.float32),
            pltpu.VMEM((zr, VW), jnp.float32),
            pltpu.VMEM((EB,), jnp.int32),
            pltpu.VMEM((1, EB), jnp.int32),
            pltpu.VMEM_SHARED((an_, VW), jnp.float32),
            pltpu.SemaphoreType.DMA,
        ],
    )
    def sc_fused(vu, src, dst, part, rows, zacc, gidx, sidx, acc_sp, sem):
        core = lax.axis_index("c")
        sub = lax.axis_index("s")
        tbase = (core * N_TILES + sub) * et
        row0 = sub * rpt
        fz = jnp.zeros((16,), jnp.float32)

        @pl.loop(0, zr)
        def _(r):
            for kk in range(VW // 16):
                zacc[r, pl.ds(16 * kk, 16)] = fz

        @pl.loop(0, reps)
        def _(kk):
            pltpu.sync_copy(zacc, acc_sp.at[pl.ds(row0 + kk * zr, zr)])

        plsc.subcore_barrier()

        @pl.loop(0, nb)
        def _(b):
            off = tbase + b * EB
            pltpu.sync_copy(src.at[pl.ds(off, EB)], gidx)
            pltpu.sync_copy(dst.at[pl.ds(off, EB)], sidx.at[0])
            pltpu.async_copy(vu.at[gidx], rows, sem).wait()
            pltpu.sync_copy(rows, acc_sp.at[sidx.at[0]], add=True)

        plsc.subcore_barrier()
        pltpu.sync_copy(acc_sp.at[pl.ds(row0, rpt)],
                        part.at[core, 0, pl.ds(row0, rpt)])

    return sc_fused


_sc_chunk_se = _make_sc_chunk_kernel(SE_AN, SE_EP)
_sc_fused_se = _make_sc_fused_kernel(FU_AN, SE_EP)
_sc_fused_ek = _make_sc_fused_kernel(FU_AN, EK_EP)


def _conv_chunk(part_ref):
    pp = part_ref[...]
    num = pp[0] + pp[1]
    full = jnp.concatenate([num[0], num[1], num[2][:, :KD - 2 * CW]], axis=-1)
    den = num[2][:, KD - 2 * CW:KD - 2 * CW + 1]
    return full / jnp.where(den == 0.0, 1.0, den)


def _conv_fused(part_ref):
    pp = part_ref[...]
    s = pp[0, 0] + pp[1, 0]
    num = s[:, :KD]
    den = s[:, KD:KD + 1]
    return num / jnp.where(den == 0.0, 1.0, den)


def _combine_stu_body(emb_ref, part_ref, out_ref):
    out_ref[...] = emb_ref[...] + _conv_chunk(part_ref)


def _combine_stu(emb, part, bn=400):
    n = emb.shape[0]
    grid = (n // bn,)
    return pl.pallas_call(
        _combine_stu_body,
        grid=grid,
        in_specs=[
            pl.BlockSpec((bn, KD), lambda i: (i, 0)),
            pl.BlockSpec((2, 3, bn, CW), lambda i: (0, 0, i, 0)),
        ],
        out_specs=pl.BlockSpec((bn, KD), lambda i: (i, 0)),
        out_shape=jax.ShapeDtypeStruct((n, KD), jnp.float32),
    )(emb, part)


def _combine_add_fused_body(emb_ref, part_ref, out_ref):
    out_ref[...] = emb_ref[...] + _conv_fused(part_ref)


def _combine_add_fused(emb, part, row_off, bn):
    n = emb.shape[0]
    grid = (n // bn,)
    ob = row_off // bn
    return pl.pallas_call(
        _combine_add_fused_body,
        grid=grid,
        in_specs=[
            pl.BlockSpec((bn, KD), lambda i: (i, 0)),
            pl.BlockSpec((2, 1, bn, VW), lambda i, ob=ob: (0, 0, ob + i, 0)),
        ],
        out_specs=pl.BlockSpec((bn, KD), lambda i: (i, 0)),
        out_shape=jax.ShapeDtypeStruct((n, KD), jnp.float32),
    )(emb, part)


def _combine_exer_body(emb_ref, p0_ref, p1_ref,
                       wa0_ref, wa1_ref, ba0_ref, ba1_ref, out_ref):
    emb = emb_ref[...]
    c0 = _conv_fused(p0_ref)
    c1 = _conv_fused(p1_ref)
    wa0 = wa0_ref[...]
    wa1 = wa1_ref[...]
    s0 = (jnp.sum(emb * wa0[0:1, :], axis=1, keepdims=True)
          + jnp.sum(c0 * wa0[1:2, :], axis=1, keepdims=True) + ba0_ref[0])
    s1 = (jnp.sum(emb * wa1[0:1, :], axis=1, keepdims=True)
          + jnp.sum(c1 * wa1[1:2, :], axis=1, keepdims=True) + ba1_ref[0])
    m = jnp.maximum(s0, s1)
    e0 = jnp.exp(s0 - m)
    e1 = jnp.exp(s1 - m)
    out_ref[...] = emb + (e0 * c0 + e1 * c1) / (e0 + e1)


def _combine_exer(emb, p0, p1, wa0, wa1, ba0, ba1, bn=400):
    n = emb.shape[0]
    grid = (n // bn,)
    return pl.pallas_call(
        _combine_exer_body,
        grid=grid,
        in_specs=[
            pl.BlockSpec((bn, KD), lambda i: (i, 0)),
            pl.BlockSpec((2, 1, bn, VW), lambda i: (0, 0, i, 0)),
            pl.BlockSpec((2, 1, bn, VW), lambda i: (0, 0, i, 0)),
            pl.BlockSpec((2, KD), lambda i: (0, 0)),
            pl.BlockSpec((2, KD), lambda i: (0, 0)),
            pl.BlockSpec(memory_space=pltpu.SMEM),
            pl.BlockSpec(memory_space=pltpu.SMEM),
        ],
        out_specs=pl.BlockSpec((bn, KD), lambda i: (i, 0)),
        out_shape=jax.ShapeDtypeStruct((n, KD), jnp.float32),
    )(emb, p0, p1, wa0, wa1, ba0, ba1)


def _pad_rows(x, np_):
    return jnp.pad(x, ((0, np_ - x.shape[0]), (0, 0)))


def _pad1(x, ep, cv=0):
    return jnp.pad(x, (0, ep - x.shape[0]), constant_values=cv)


def _acol(a):
    return a[:KD].reshape(KD, 1)


def kernel(stu_emb, exer_emb, kn_emb, W_sfe, a_sfe, W_efs, a_efs, W_efk,
           a_efk, W_kfe, a_kfe, Wa_s0, ba_s0, Wa_e0, ba_e0, Wa_e1, ba_e1,
           Wa_k0, ba_k0, s_from_e_edge_index, e_from_s_edge_index,
           e_from_k_edge_index, k_from_e_edge_index):
    h_se = _pad_rows(jnp.concatenate([exer_emb, stu_emb], axis=0), SE_NP)
    h_ek = _pad_rows(jnp.concatenate([exer_emb, kn_emb], axis=0), EK_NP)

    vu_sfe, vu_efs = _mm_pair(
        h_se, W_sfe.T, _acol(a_sfe), W_efs.T, _acol(a_efs))
    vu_efk, vu_kfe = _mm_pair(
        h_ek, W_efk.T, _acol(a_efk), W_kfe.T, _acol(a_kfe))

    src_sfe = _pad1(s_from_e_edge_index[0], SE_EP)
    dst_sfe = _pad1(jnp.minimum(s_from_e_edge_index[1], S_N), SE_EP, S_N)
    src_efs = _pad1(e_from_s_edge_index[0], SE_EP)
    d = e_from_s_edge_index[1]
    dst_efs = _pad1(jnp.where(d >= S_N, d - S_N, X_N), SE_EP, X_N)
    src_efk = _pad1(e_from_k_edge_index[0], EK_EP)
    dst_efk = _pad1(e_from_k_edge_index[1], EK_EP, EK_N)
    src_kfe = _pad1(k_from_e_edge_index[0], EK_EP)
    dst_kfe = _pad1(k_from_e_edge_index[1], EK_EP, EK_N)

    part_sfe = _sc_chunk_se(
        vu_sfe.reshape(SE_NP * 3, CW), src_sfe, dst_sfe)
    part_efs = _sc_fused_se(vu_efs, src_efs, dst_efs)
    part_efk = _sc_fused_ek(vu_efk, src_efk, dst_efk)
    part_kfe = _sc_fused_ek(vu_kfe, src_kfe, dst_kfe)

    ult_stu = _combine_stu(stu_emb, part_sfe)
    ult_kn = _combine_add_fused(kn_emb, part_kfe, X_N, 16)
    ult_exer = _combine_exer(
        exer_emb, part_efs, part_efk,
        Wa_e0.reshape(2, KD), Wa_e1.reshape(2, KD), ba_e0, ba_e1)

    return (ult_stu, ult_exer, ult_kn)
